# baseline (speedup 1.0000x reference)
import functools

import jax
import jax.numpy as jnp
from jax import lax
from jax.experimental import pallas as pl
from jax.experimental.pallas import tpu as pltpu
from jax.experimental.pallas import tpu_sc as plsc

N = 10000
E = 320000
NC = 2
NS = 16
NW = NC * NS
EPW = E // NW
CH = 1000
RCH = 256
RNCH = E // RCH
RJMAX = -(-RNCH // NW)
NPAD = 10240
SPT = NPAD // NS

_mesh = plsc.VectorSubcoreMesh(core_axis_name="c", subcore_axis_name="s")


def _wid():
    return lax.axis_index("s") * NC + lax.axis_index("c")


def _deg_body(dst0, dst1, ones_hbm, zeros_hbm, out, acc0, acc1, ones_v,
              idx0, idx1, sem):
    cid = lax.axis_index("c")
    sid = lax.axis_index("s")
    wid = _wid()
    pltpu.sync_copy(zeros_hbm.at[pl.ds(0, SPT)], acc0.at[pl.ds(sid * SPT, SPT)])
    pltpu.sync_copy(zeros_hbm.at[pl.ds(0, SPT)], acc1.at[pl.ds(sid * SPT, SPT)])
    pltpu.sync_copy(ones_hbm.at[pl.ds(0, CH)], ones_v)
    plsc.subcore_barrier()

    def it(k, carry):
        base = wid * EPW + k * CH
        pltpu.sync_copy(dst0.at[pl.ds(base, CH)], idx0)
        pltpu.sync_copy(dst1.at[pl.ds(base, CH)], idx1)
        pltpu.sync_copy(ones_v, acc0.at[idx0], add=True)
        pltpu.sync_copy(ones_v, acc1.at[idx1], add=True)
        return carry

    lax.fori_loop(0, EPW // CH, it, 0)
    plsc.subcore_barrier()
    pltpu.sync_copy(acc0.at[pl.ds(sid * SPT, SPT)],
                    out.at[pl.ds(cid * NPAD + sid * SPT, SPT)])
    pltpu.sync_copy(acc1.at[pl.ds(sid * SPT, SPT)],
                    out.at[pl.ds((NC + cid) * NPAD + sid * SPT, SPT)])


_deg_kernel = pl.kernel(
    _deg_body,
    out_type=jax.ShapeDtypeStruct((2 * NC * NPAD,), jnp.float32),
    mesh=_mesh,
    scratch_types=[
        pltpu.VMEM_SHARED((NPAD,), jnp.float32),
        pltpu.VMEM_SHARED((NPAD,), jnp.float32),
        pltpu.VMEM((CH,), jnp.float32),
        pltpu.VMEM((CH,), jnp.int32),
        pltpu.VMEM((CH,), jnp.int32),
        pltpu.SemaphoreType.DMA,
    ],
)


def _rowagg_body(table, src, dst, zeros_hbm, out, acc, idx_s, idx_d, rows,
                 sem):
    cid = lax.axis_index("c")
    sid = lax.axis_index("s")
    wid = _wid()
    pltpu.sync_copy(zeros_hbm.at[pl.ds(0, SPT)],
                    acc.at[pl.ds(sid * SPT, SPT)])
    plsc.subcore_barrier()

    def it(j, carry):
        chunk = j * NW + wid

        @pl.when(chunk < RNCH)
        def _():
            base = chunk * RCH
            pltpu.sync_copy(src.at[pl.ds(base, RCH)], idx_s)
            pltpu.sync_copy(dst.at[pl.ds(base, RCH)], idx_d)
            pltpu.async_copy(table.at[idx_s], rows, sem).wait()
            pltpu.sync_copy(rows, acc.at[idx_d], add=True)

        return carry

    lax.fori_loop(0, RJMAX, it, 0)
    plsc.subcore_barrier()
    pltpu.sync_copy(acc.at[pl.ds(sid * SPT, SPT)],
                    out.at[cid, pl.ds(sid * SPT, SPT)])


def _make_rowagg(D):
    return pl.kernel(
        _rowagg_body,
        out_type=jax.ShapeDtypeStruct((NC, NPAD, D), jnp.float32),
        mesh=_mesh,
        scratch_types=[
            pltpu.VMEM_SHARED((NPAD, D), jnp.float32),
            pltpu.VMEM((RCH,), jnp.int32),
            pltpu.VMEM((RCH,), jnp.int32),
            pltpu.VMEM((RCH, D), jnp.float32),
            pltpu.SemaphoreType.DMA,
        ],
    )


_rowagg128 = _make_rowagg(128)

GCH = 256
GNCH = E // GCH
GJMAX = -(-GNCH // NW)


def _gat_body(h, ed, es, src, dst, zrows, z1d, outn, outd,
              accn, accd, idx_s, idx_d, rows, edv, esv, exv, sem):
    cid = lax.axis_index("c")
    sid = lax.axis_index("s")
    wid = _wid()
    pltpu.sync_copy(zrows.at[pl.ds(0, SPT)], accn.at[pl.ds(sid * SPT, SPT)])
    pltpu.sync_copy(z1d.at[pl.ds(0, SPT)], accd.at[pl.ds(sid * SPT, SPT)])
    plsc.subcore_barrier()

    def it(j, carry):
        chunk = j * NW + wid

        @pl.when(chunk < GNCH)
        def _():
            base = chunk * GCH
            pltpu.sync_copy(src.at[pl.ds(base, GCH)], idx_s)
            pltpu.sync_copy(dst.at[pl.ds(base, GCH)], idx_d)
            pltpu.async_copy(es.at[idx_s], esv, sem).wait()
            pltpu.async_copy(ed.at[idx_d], edv, sem).wait()
            pltpu.async_copy(h.at[idx_s], rows, sem).wait()

            for g in range(GCH // 16):
                e = edv[pl.ds(g * 16, 16)] + esv[pl.ds(g * 16, 16)]
                ex = jnp.exp(jnp.maximum(e, 0.2 * e))
                exv[pl.ds(g * 16, 16)] = ex
                for jj in range(16):
                    i = g * 16 + jj
                    b = jnp.full((16,), ex[jj])
                    for q in range(4):
                        rows[i, pl.ds(q * 16, 16)] = (
                            rows[i, pl.ds(q * 16, 16)] * b)

            pltpu.sync_copy(exv, accd.at[idx_d], add=True)
            pltpu.sync_copy(rows, accn.at[idx_d], add=True)

        return carry

    lax.fori_loop(0, GJMAX, it, 0)
    plsc.subcore_barrier()
    pltpu.sync_copy(accn.at[pl.ds(sid * SPT, SPT)],
                    outn.at[cid, pl.ds(sid * SPT, SPT)])
    pltpu.sync_copy(accd.at[pl.ds(sid * SPT, SPT)],
                    outd.at[pl.ds(cid * NPAD + sid * SPT, SPT)])


_gat_kernel = pl.kernel(
    _gat_body,
    out_type=(jax.ShapeDtypeStruct((NC, NPAD, 128), jnp.float32),
              jax.ShapeDtypeStruct((NC * NPAD,), jnp.float32)),
    mesh=_mesh,
    scratch_types=[
        pltpu.VMEM_SHARED((NPAD, 128), jnp.float32),
        pltpu.VMEM_SHARED((NPAD,), jnp.float32),
        pltpu.VMEM((GCH,), jnp.int32),
        pltpu.VMEM((GCH,), jnp.int32),
        pltpu.VMEM((GCH, 128), jnp.float32),
        pltpu.VMEM((GCH,), jnp.float32),
        pltpu.VMEM((GCH,), jnp.float32),
        pltpu.VMEM((GCH,), jnp.float32),
        pltpu.SemaphoreType.DMA,
    ],
)


def _prep_body(deg4, x, x1, xs, xs1, dinv0, dinv1, inv0, inv1):
    i0 = deg4[0] + deg4[1]
    i1 = deg4[2] + deg4[3]
    dv0 = 1.0 / jnp.sqrt(i0 + 1.0)
    dv1 = 1.0 / jnp.sqrt(i1 + 1.0)
    dinv0[...] = dv0
    dinv1[...] = dv1
    inv0[...] = 1.0 / jnp.maximum(i0, 1.0)
    inv1[...] = 1.0 / jnp.maximum(i1, 1.0)
    xs[:N, :] = x[...] * dv0[:N].reshape(N, 1)
    xs[N:, :] = jnp.zeros((NPAD - N, 128), jnp.float32)
    xs1[:N, :] = x1[...] * dv1[:N].reshape(N, 1)
    xs1[N:, :] = jnp.zeros((NPAD - N, 128), jnp.float32)


_prep_tc = pl.pallas_call(
    _prep_body,
    out_shape=(
        jax.ShapeDtypeStruct((NPAD, 128), jnp.float32),
        jax.ShapeDtypeStruct((NPAD, 128), jnp.float32),
        jax.ShapeDtypeStruct((NPAD,), jnp.float32),
        jax.ShapeDtypeStruct((NPAD,), jnp.float32),
        jax.ShapeDtypeStruct((NPAD,), jnp.float32),
        jax.ShapeDtypeStruct((NPAD,), jnp.float32),
    ),
)


def _gcn_mm_body(parts, xs, dinv, W, b, out):
    agg = parts[0] + parts[1] + xs[...]
    scaled = agg * dinv[...].reshape(NPAD, 1)
    out[...] = jnp.dot(scaled, W[...],
                       preferred_element_type=jnp.float32) + b[...]


_gcn_mm = pl.pallas_call(
    _gcn_mm_body,
    out_shape=jax.ShapeDtypeStruct((NPAD, 256), jnp.float32),
)


def _sagepre1_body(xa, xb, Wn, Ws, p, q):
    x0 = jax.nn.relu(xa[...]) + jax.nn.relu(xb[...])
    p[...] = jnp.dot(x0, Wn[...], preferred_element_type=jnp.float32)
    q[...] = jnp.dot(x0, Ws[...], preferred_element_type=jnp.float32)


def _sagepre2_body(xb, Wn, Ws, p, q):
    t = jax.nn.relu(xb[...]) + xb[...]
    p[...] = jnp.dot(t, Wn[...], preferred_element_type=jnp.float32)
    q[...] = jnp.dot(t, Ws[...], preferred_element_type=jnp.float32)


_sage_out = (jax.ShapeDtypeStruct((NPAD, 128), jnp.float32),
             jax.ShapeDtypeStruct((NPAD, 128), jnp.float32))
_sagepre1 = pl.pallas_call(_sagepre1_body, out_shape=_sage_out)
_sagepre2 = pl.pallas_call(_sagepre2_body, out_shape=_sage_out)


def _gatpre_body(parts, inv, q, bn, W, Wb, a, ab, h128, ed, es):
    xc = (parts[0] + parts[1]) * inv[...].reshape(NPAD, 1) + bn[...] + q[...]
    h = jnp.dot(xc, W[...], preferred_element_type=jnp.float32) + Wb[...]
    h128[:, :64] = h
    h128[:, 64:] = jnp.zeros((NPAD, 64), jnp.float32)
    ed[...] = jnp.dot(h, a[:64, 0], preferred_element_type=jnp.float32) + ab[...]
    es[...] = jnp.dot(h, a[64:, 0], preferred_element_type=jnp.float32)


_gatpre = pl.pallas_call(
    _gatpre_body,
    out_shape=(
        jax.ShapeDtypeStruct((NPAD, 128), jnp.float32),
        jax.ShapeDtypeStruct((NPAD,), jnp.float32),
        jax.ShapeDtypeStruct((NPAD,), jnp.float32),
    ),
)


def _final_body(n1, dn1, n2, dn2, W1, b1, g, bta, W2, b2, out):
    def branch(n, dn):
        num = n[0, :, :64] + n[1, :, :64]
        den = dn[0] + dn[1]
        deni = jnp.where(den > 0, 1.0 / den, 0.0).reshape(NPAD, 1)
        xc = jax.nn.relu(num * deni)
        z = jnp.dot(xc, W1[...], preferred_element_type=jnp.float32) + b1[...]
        mu = jnp.mean(z, axis=-1, keepdims=True)
        var = jnp.mean((z - mu) * (z - mu), axis=-1, keepdims=True)
        z = (z - mu) / jnp.sqrt(var + 1e-5) * g[...] + bta[...]
        z = jnp.dot(z, W2[...], preferred_element_type=jnp.float32) + b2[...]
        return jnp.sum(z[:N, :])

    out[...] = (branch(n1, dn1) + branch(n2, dn2)).reshape(1, 1)


_final_tc = pl.pallas_call(
    _final_body,
    out_shape=jax.ShapeDtypeStruct((1, 1), jnp.float32),
)


def kernel(x, x1, edge_index, edge_index1, gcn1_W, gcn1_b, gcn2_W, gcn2_b,
           sage1_Wn, sage1_b, sage1_Ws, sage2_Wn, sage2_b, sage2_Ws,
           gat1_W, gat1_Wb, gat1_a, gat1_ab, gat2_W, gat2_Wb, gat2_a, gat2_ab,
           mlp_W1, mlp_b1, ln_g, ln_b, mlp_W2, mlp_b2):
    ei = edge_index.astype(jnp.int32)
    ei1 = edge_index1.astype(jnp.int32)
    s0, d0 = ei[0], ei[1]
    s1, d1 = ei1[0], ei1[1]

    ones_ch = jnp.ones((CH,), jnp.float32)
    zeros_1d = jnp.zeros((SPT,), jnp.float32)
    zeros_rows = jnp.zeros((SPT, 128), jnp.float32)

    deg4 = _deg_kernel(d0, d1, ones_ch, zeros_1d).reshape(2 * NC, NPAD)
    xs, xs1, dinv0, dinv1, inv0, inv1 = _prep_tc(deg4, x, x1)

    A0 = _rowagg128(xs, s0, d0, zeros_rows)
    A1 = _rowagg128(xs1, s1, d1, zeros_rows)
    xa = _gcn_mm(A0, xs, dinv0, gcn1_W, gcn1_b)
    xb = _gcn_mm(A1, xs1, dinv1, gcn2_W, gcn2_b)

    p1, q1 = _sagepre1(xa, xb, sage1_Wn, sage1_Ws)
    p2, q2 = _sagepre2(xb, sage2_Wn, sage2_Ws)
    P1 = _rowagg128(p1, s0, d0, zeros_rows)
    P2 = _rowagg128(p2, s1, d1, zeros_rows)

    h1, ed1, es1 = _gatpre(P1, inv0, q1, sage1_b, gat1_W, gat1_Wb,
                           gat1_a, gat1_ab)
    h2, ed2, es2 = _gatpre(P2, inv1, q2, sage2_b, gat2_W, gat2_Wb,
                           gat2_a, gat2_ab)

    n1, dn1 = _gat_kernel(h1, ed1, es1, s0, d0, zeros_rows, zeros_1d)
    n2, dn2 = _gat_kernel(h2, ed2, es2, s1, d1, zeros_rows, zeros_1d)

    total = _final_tc(n1, dn1.reshape(NC, NPAD), n2, dn2.reshape(NC, NPAD),
                      mlp_W1, mlp_b1, ln_g, ln_b, mlp_W2, mlp_b2)
    return total / (2 * N * 64)


# speedup vs baseline: 1.1656x; 1.1656x over previous
import functools

import jax
import jax.numpy as jnp
from jax import lax
from jax.experimental import pallas as pl
from jax.experimental.pallas import tpu as pltpu
from jax.experimental.pallas import tpu_sc as plsc

N = 10000
E = 320000
NC = 2
NS = 16
NW = NC * NS
EPW = E // NW
CH = 1000
RCH = 256
RNCH = E // RCH
RJMAX = -(-RNCH // NW)
NPAD = 10240
SPT = NPAD // NS

_mesh = plsc.VectorSubcoreMesh(core_axis_name="c", subcore_axis_name="s")


def _wid():
    return lax.axis_index("s") * NC + lax.axis_index("c")


def _deg_body(dst0, dst1, ones_hbm, zeros_hbm, out, acc0, acc1, ones_v,
              idx0, idx1, sem):
    cid = lax.axis_index("c")
    sid = lax.axis_index("s")
    wid = _wid()
    pltpu.sync_copy(zeros_hbm.at[pl.ds(0, SPT)], acc0.at[pl.ds(sid * SPT, SPT)])
    pltpu.sync_copy(zeros_hbm.at[pl.ds(0, SPT)], acc1.at[pl.ds(sid * SPT, SPT)])
    pltpu.sync_copy(ones_hbm.at[pl.ds(0, CH)], ones_v)
    plsc.subcore_barrier()

    def it(k, carry):
        base = wid * EPW + k * CH
        pltpu.sync_copy(dst0.at[pl.ds(base, CH)], idx0)
        pltpu.sync_copy(dst1.at[pl.ds(base, CH)], idx1)
        pltpu.sync_copy(ones_v, acc0.at[idx0], add=True)
        pltpu.sync_copy(ones_v, acc1.at[idx1], add=True)
        return carry

    lax.fori_loop(0, EPW // CH, it, 0)
    plsc.subcore_barrier()
    pltpu.sync_copy(acc0.at[pl.ds(sid * SPT, SPT)],
                    out.at[pl.ds(cid * NPAD + sid * SPT, SPT)])
    pltpu.sync_copy(acc1.at[pl.ds(sid * SPT, SPT)],
                    out.at[pl.ds((NC + cid) * NPAD + sid * SPT, SPT)])


_deg_kernel = pl.kernel(
    _deg_body,
    out_type=jax.ShapeDtypeStruct((2 * NC * NPAD,), jnp.float32),
    mesh=_mesh,
    scratch_types=[
        pltpu.VMEM_SHARED((NPAD,), jnp.float32),
        pltpu.VMEM_SHARED((NPAD,), jnp.float32),
        pltpu.VMEM((CH,), jnp.float32),
        pltpu.VMEM((CH,), jnp.int32),
        pltpu.VMEM((CH,), jnp.int32),
        pltpu.SemaphoreType.DMA,
    ],
)


RCH2 = 160
EPT = E // NS
NCHT = EPT // RCH2


def _rowagg2_body(t0, t1, sa0, da0, sa1, da1, zrows, out, acc,
                  ixs0, ixd0, ixs1, ixd1, rows0, rows1, gsem0, gsem1):
    cid = lax.axis_index("c")
    sid = lax.axis_index("s")
    pltpu.sync_copy(zrows.at[pl.ds(0, SPT)], acc.at[pl.ds(sid * SPT, SPT)])
    plsc.subcore_barrier()

    def graph_loop(table, src, dst):
        base0 = sid * EPT

        def load_idx(c, ixs, ixd):
            pltpu.sync_copy(src.at[pl.ds(base0 + c * RCH2, RCH2)], ixs)
            pltpu.sync_copy(dst.at[pl.ds(base0 + c * RCH2, RCH2)], ixd)

        load_idx(0, ixs0, ixd0)
        pltpu.async_copy(table.at[ixs0], rows0, gsem0)

        def body(k, carry):
            c1 = 2 * k + 1
            c2 = 2 * k + 2

            @pl.when(c1 < NCHT)
            def _():
                load_idx(c1, ixs1, ixd1)
                pltpu.async_copy(table.at[ixs1], rows1, gsem1)

            pltpu.make_async_copy(table.at[ixs0], rows0, gsem0).wait()
            pltpu.sync_copy(rows0, acc.at[ixd0], add=True)

            @pl.when(c2 < NCHT)
            def _():
                load_idx(c2, ixs0, ixd0)
                pltpu.async_copy(table.at[ixs0], rows0, gsem0)

            @pl.when(c1 < NCHT)
            def _():
                pltpu.make_async_copy(table.at[ixs1], rows1, gsem1).wait()
                pltpu.sync_copy(rows1, acc.at[ixd1], add=True)

            return carry

        lax.fori_loop(0, (NCHT + 1) // 2, body, 0)

    @pl.when(cid == 0)
    def _():
        graph_loop(t0, sa0, da0)

    @pl.when(cid == 1)
    def _():
        graph_loop(t1, sa1, da1)

    plsc.subcore_barrier()
    pltpu.sync_copy(acc.at[pl.ds(sid * SPT, SPT)],
                    out.at[cid, pl.ds(sid * SPT, SPT)])


_rowagg2 = pl.kernel(
    _rowagg2_body,
    out_type=jax.ShapeDtypeStruct((NC, NPAD, 128), jnp.float32),
    mesh=_mesh,
    scratch_types=[
        pltpu.VMEM_SHARED((NPAD, 128), jnp.float32),
        pltpu.VMEM((RCH2,), jnp.int32),
        pltpu.VMEM((RCH2,), jnp.int32),
        pltpu.VMEM((RCH2,), jnp.int32),
        pltpu.VMEM((RCH2,), jnp.int32),
        pltpu.VMEM((RCH2, 128), jnp.float32),
        pltpu.VMEM((RCH2, 128), jnp.float32),
        pltpu.SemaphoreType.DMA,
        pltpu.SemaphoreType.DMA,
    ],
)

GCH = 256
GNCH = E // GCH
GJMAX = -(-GNCH // NW)


def _gat_body(h, ed, es, src, dst, zrows, z1d, outn, outd,
              accn, accd, idx_s, idx_d, rows, edv, esv, exv, sem):
    cid = lax.axis_index("c")
    sid = lax.axis_index("s")
    wid = _wid()
    pltpu.sync_copy(zrows.at[pl.ds(0, SPT)], accn.at[pl.ds(sid * SPT, SPT)])
    pltpu.sync_copy(z1d.at[pl.ds(0, SPT)], accd.at[pl.ds(sid * SPT, SPT)])
    plsc.subcore_barrier()

    def it(j, carry):
        chunk = j * NW + wid

        @pl.when(chunk < GNCH)
        def _():
            base = chunk * GCH
            pltpu.sync_copy(src.at[pl.ds(base, GCH)], idx_s)
            pltpu.sync_copy(dst.at[pl.ds(base, GCH)], idx_d)
            pltpu.async_copy(es.at[idx_s], esv, sem).wait()
            pltpu.async_copy(ed.at[idx_d], edv, sem).wait()
            pltpu.async_copy(h.at[idx_s], rows, sem).wait()

            for g in range(GCH // 16):
                e = edv[pl.ds(g * 16, 16)] + esv[pl.ds(g * 16, 16)]
                ex = jnp.exp(jnp.maximum(e, 0.2 * e))
                exv[pl.ds(g * 16, 16)] = ex
                for jj in range(16):
                    i = g * 16 + jj
                    b = jnp.full((16,), ex[jj])
                    for q in range(4):
                        rows[i, pl.ds(q * 16, 16)] = (
                            rows[i, pl.ds(q * 16, 16)] * b)

            pltpu.sync_copy(exv, accd.at[idx_d], add=True)
            pltpu.sync_copy(rows, accn.at[idx_d], add=True)

        return carry

    lax.fori_loop(0, GJMAX, it, 0)
    plsc.subcore_barrier()
    pltpu.sync_copy(accn.at[pl.ds(sid * SPT, SPT)],
                    outn.at[cid, pl.ds(sid * SPT, SPT)])
    pltpu.sync_copy(accd.at[pl.ds(sid * SPT, SPT)],
                    outd.at[pl.ds(cid * NPAD + sid * SPT, SPT)])


_gat_kernel = pl.kernel(
    _gat_body,
    out_type=(jax.ShapeDtypeStruct((NC, NPAD, 128), jnp.float32),
              jax.ShapeDtypeStruct((NC * NPAD,), jnp.float32)),
    mesh=_mesh,
    scratch_types=[
        pltpu.VMEM_SHARED((NPAD, 128), jnp.float32),
        pltpu.VMEM_SHARED((NPAD,), jnp.float32),
        pltpu.VMEM((GCH,), jnp.int32),
        pltpu.VMEM((GCH,), jnp.int32),
        pltpu.VMEM((GCH, 128), jnp.float32),
        pltpu.VMEM((GCH,), jnp.float32),
        pltpu.VMEM((GCH,), jnp.float32),
        pltpu.VMEM((GCH,), jnp.float32),
        pltpu.SemaphoreType.DMA,
    ],
)


def _prep_body(deg4, x, x1, xs, xs1, dinv0, dinv1, inv0, inv1):
    i0 = deg4[0] + deg4[1]
    i1 = deg4[2] + deg4[3]
    dv0 = 1.0 / jnp.sqrt(i0 + 1.0)
    dv1 = 1.0 / jnp.sqrt(i1 + 1.0)
    dinv0[...] = dv0
    dinv1[...] = dv1
    inv0[...] = 1.0 / jnp.maximum(i0, 1.0)
    inv1[...] = 1.0 / jnp.maximum(i1, 1.0)
    xs[:N, :] = x[...] * dv0[:N].reshape(N, 1)
    xs[N:, :] = jnp.zeros((NPAD - N, 128), jnp.float32)
    xs1[:N, :] = x1[...] * dv1[:N].reshape(N, 1)
    xs1[N:, :] = jnp.zeros((NPAD - N, 128), jnp.float32)


_prep_tc = pl.pallas_call(
    _prep_body,
    out_shape=(
        jax.ShapeDtypeStruct((NPAD, 128), jnp.float32),
        jax.ShapeDtypeStruct((NPAD, 128), jnp.float32),
        jax.ShapeDtypeStruct((NPAD,), jnp.float32),
        jax.ShapeDtypeStruct((NPAD,), jnp.float32),
        jax.ShapeDtypeStruct((NPAD,), jnp.float32),
        jax.ShapeDtypeStruct((NPAD,), jnp.float32),
    ),
)


def _gcn_mm_body(agg_in, xs, dinv, W, b, out):
    agg = agg_in[...] + xs[...]
    scaled = agg * dinv[...].reshape(NPAD, 1)
    out[...] = jnp.dot(scaled, W[...],
                       preferred_element_type=jnp.float32) + b[...]


_gcn_mm = pl.pallas_call(
    _gcn_mm_body,
    out_shape=jax.ShapeDtypeStruct((NPAD, 256), jnp.float32),
)


def _sagepre1_body(xa, xb, Wn, Ws, p, q):
    x0 = jax.nn.relu(xa[...]) + jax.nn.relu(xb[...])
    p[...] = jnp.dot(x0, Wn[...], preferred_element_type=jnp.float32)
    q[...] = jnp.dot(x0, Ws[...], preferred_element_type=jnp.float32)


def _sagepre2_body(xb, Wn, Ws, p, q):
    t = jax.nn.relu(xb[...]) + xb[...]
    p[...] = jnp.dot(t, Wn[...], preferred_element_type=jnp.float32)
    q[...] = jnp.dot(t, Ws[...], preferred_element_type=jnp.float32)


_sage_out = (jax.ShapeDtypeStruct((NPAD, 128), jnp.float32),
             jax.ShapeDtypeStruct((NPAD, 128), jnp.float32))
_sagepre1 = pl.pallas_call(_sagepre1_body, out_shape=_sage_out)
_sagepre2 = pl.pallas_call(_sagepre2_body, out_shape=_sage_out)


def _gatpre_body(agg_in, inv, q, bn, W, Wb, a, ab, h128, ed, es):
    xc = agg_in[...] * inv[...].reshape(NPAD, 1) + bn[...] + q[...]
    h = jnp.dot(xc, W[...], preferred_element_type=jnp.float32) + Wb[...]
    h128[:, :64] = h
    h128[:, 64:] = jnp.zeros((NPAD, 64), jnp.float32)
    ed[...] = jnp.dot(h, a[:64, 0], preferred_element_type=jnp.float32) + ab[...]
    es[...] = jnp.dot(h, a[64:, 0], preferred_element_type=jnp.float32)


_gatpre = pl.pallas_call(
    _gatpre_body,
    out_shape=(
        jax.ShapeDtypeStruct((NPAD, 128), jnp.float32),
        jax.ShapeDtypeStruct((NPAD,), jnp.float32),
        jax.ShapeDtypeStruct((NPAD,), jnp.float32),
    ),
)


def _final_body(n1, dn1, n2, dn2, W1, b1, g, bta, W2, b2, out):
    def branch(n, dn):
        num = n[0, :, :64] + n[1, :, :64]
        den = dn[0] + dn[1]
        deni = jnp.where(den > 0, 1.0 / den, 0.0).reshape(NPAD, 1)
        xc = jax.nn.relu(num * deni)
        z = jnp.dot(xc, W1[...], preferred_element_type=jnp.float32) + b1[...]
        mu = jnp.mean(z, axis=-1, keepdims=True)
        var = jnp.mean((z - mu) * (z - mu), axis=-1, keepdims=True)
        z = (z - mu) / jnp.sqrt(var + 1e-5) * g[...] + bta[...]
        z = jnp.dot(z, W2[...], preferred_element_type=jnp.float32) + b2[...]
        return jnp.sum(z[:N, :])

    out[...] = (branch(n1, dn1) + branch(n2, dn2)).reshape(1, 1)


_final_tc = pl.pallas_call(
    _final_body,
    out_shape=jax.ShapeDtypeStruct((1, 1), jnp.float32),
)


def kernel(x, x1, edge_index, edge_index1, gcn1_W, gcn1_b, gcn2_W, gcn2_b,
           sage1_Wn, sage1_b, sage1_Ws, sage2_Wn, sage2_b, sage2_Ws,
           gat1_W, gat1_Wb, gat1_a, gat1_ab, gat2_W, gat2_Wb, gat2_a, gat2_ab,
           mlp_W1, mlp_b1, ln_g, ln_b, mlp_W2, mlp_b2):
    ei = edge_index.astype(jnp.int32)
    ei1 = edge_index1.astype(jnp.int32)
    s0, d0 = ei[0], ei[1]
    s1, d1 = ei1[0], ei1[1]

    ones_ch = jnp.ones((CH,), jnp.float32)
    zeros_1d = jnp.zeros((SPT,), jnp.float32)
    zeros_rows = jnp.zeros((SPT, 128), jnp.float32)

    deg4 = _deg_kernel(d0, d1, ones_ch, zeros_1d).reshape(2 * NC, NPAD)
    xs, xs1, dinv0, dinv1, inv0, inv1 = _prep_tc(deg4, x, x1)

    A = _rowagg2(xs, xs1, s0, d0, s1, d1, zeros_rows)
    xa = _gcn_mm(A[0], xs, dinv0, gcn1_W, gcn1_b)
    xb = _gcn_mm(A[1], xs1, dinv1, gcn2_W, gcn2_b)

    p1, q1 = _sagepre1(xa, xb, sage1_Wn, sage1_Ws)
    p2, q2 = _sagepre2(xb, sage2_Wn, sage2_Ws)
    P = _rowagg2(p1, p2, s0, d0, s1, d1, zeros_rows)

    h1, ed1, es1 = _gatpre(P[0], inv0, q1, sage1_b, gat1_W, gat1_Wb,
                           gat1_a, gat1_ab)
    h2, ed2, es2 = _gatpre(P[1], inv1, q2, sage2_b, gat2_W, gat2_Wb,
                           gat2_a, gat2_ab)

    n1, dn1 = _gat_kernel(h1, ed1, es1, s0, d0, zeros_rows, zeros_1d)
    n2, dn2 = _gat_kernel(h2, ed2, es2, s1, d1, zeros_rows, zeros_1d)

    total = _final_tc(n1, dn1.reshape(NC, NPAD), n2, dn2.reshape(NC, NPAD),
                      mlp_W1, mlp_b1, ln_g, ln_b, mlp_W2, mlp_b2)
    return total / (2 * N * 64)


# speedup vs baseline: 1.4291x; 1.2260x over previous
import functools

import jax
import jax.numpy as jnp
from jax import lax
from jax.experimental import pallas as pl
from jax.experimental.pallas import tpu as pltpu
from jax.experimental.pallas import tpu_sc as plsc

N = 10000
E = 320000
NC = 2
NS = 16
NW = NC * NS
EPW = E // NW
CH = 1000
RCH = 256
RNCH = E // RCH
RJMAX = -(-RNCH // NW)
NPAD = 10240
SPT = NPAD // NS

_mesh = plsc.VectorSubcoreMesh(core_axis_name="c", subcore_axis_name="s")


def _wid():
    return lax.axis_index("s") * NC + lax.axis_index("c")


def _deg_body(dst0, dst1, ones_hbm, zeros_hbm, out, acc0, acc1, ones_v,
              idx0, idx1, sem):
    cid = lax.axis_index("c")
    sid = lax.axis_index("s")
    wid = _wid()
    pltpu.sync_copy(zeros_hbm.at[pl.ds(0, SPT)], acc0.at[pl.ds(sid * SPT, SPT)])
    pltpu.sync_copy(zeros_hbm.at[pl.ds(0, SPT)], acc1.at[pl.ds(sid * SPT, SPT)])
    pltpu.sync_copy(ones_hbm.at[pl.ds(0, CH)], ones_v)
    plsc.subcore_barrier()

    def it(k, carry):
        base = wid * EPW + k * CH
        pltpu.sync_copy(dst0.at[pl.ds(base, CH)], idx0)
        pltpu.sync_copy(dst1.at[pl.ds(base, CH)], idx1)
        pltpu.sync_copy(ones_v, acc0.at[idx0], add=True)
        pltpu.sync_copy(ones_v, acc1.at[idx1], add=True)
        return carry

    lax.fori_loop(0, EPW // CH, it, 0)
    plsc.subcore_barrier()
    pltpu.sync_copy(acc0.at[pl.ds(sid * SPT, SPT)],
                    out.at[pl.ds(cid * NPAD + sid * SPT, SPT)])
    pltpu.sync_copy(acc1.at[pl.ds(sid * SPT, SPT)],
                    out.at[pl.ds((NC + cid) * NPAD + sid * SPT, SPT)])


_deg_kernel = pl.kernel(
    _deg_body,
    out_type=jax.ShapeDtypeStruct((2 * NC * NPAD,), jnp.float32),
    mesh=_mesh,
    scratch_types=[
        pltpu.VMEM_SHARED((NPAD,), jnp.float32),
        pltpu.VMEM_SHARED((NPAD,), jnp.float32),
        pltpu.VMEM((CH,), jnp.float32),
        pltpu.VMEM((CH,), jnp.int32),
        pltpu.VMEM((CH,), jnp.int32),
        pltpu.SemaphoreType.DMA,
    ],
)


RCH2 = 160
EPT = E // NS
NCHT = EPT // RCH2


def _rowagg2_body(t0, t1, sa0, da0, sa1, da1, zrows, out, acc,
                  ixs0, ixd0, ixs1, ixd1, rows0, rows1, gsem0, gsem1):
    cid = lax.axis_index("c")
    sid = lax.axis_index("s")
    pltpu.sync_copy(zrows.at[pl.ds(0, SPT)], acc.at[pl.ds(sid * SPT, SPT)])
    plsc.subcore_barrier()

    def graph_loop(table, src, dst):
        base0 = sid * EPT

        def load_idx(c, ixs, ixd):
            pltpu.sync_copy(src.at[pl.ds(base0 + c * RCH2, RCH2)], ixs)
            pltpu.sync_copy(dst.at[pl.ds(base0 + c * RCH2, RCH2)], ixd)

        load_idx(0, ixs0, ixd0)
        pltpu.async_copy(table.at[ixs0], rows0, gsem0)

        def body(k, carry):
            c1 = 2 * k + 1
            c2 = 2 * k + 2

            @pl.when(c1 < NCHT)
            def _():
                load_idx(c1, ixs1, ixd1)
                pltpu.async_copy(table.at[ixs1], rows1, gsem1)

            pltpu.make_async_copy(table.at[ixs0], rows0, gsem0).wait()
            pltpu.sync_copy(rows0, acc.at[ixd0], add=True)

            @pl.when(c2 < NCHT)
            def _():
                load_idx(c2, ixs0, ixd0)
                pltpu.async_copy(table.at[ixs0], rows0, gsem0)

            @pl.when(c1 < NCHT)
            def _():
                pltpu.make_async_copy(table.at[ixs1], rows1, gsem1).wait()
                pltpu.sync_copy(rows1, acc.at[ixd1], add=True)

            return carry

        lax.fori_loop(0, (NCHT + 1) // 2, body, 0)

    @pl.when(cid == 0)
    def _():
        graph_loop(t0, sa0, da0)

    @pl.when(cid == 1)
    def _():
        graph_loop(t1, sa1, da1)

    plsc.subcore_barrier()
    pltpu.sync_copy(acc.at[pl.ds(sid * SPT, SPT)],
                    out.at[cid, pl.ds(sid * SPT, SPT)])


_rowagg2 = pl.kernel(
    _rowagg2_body,
    out_type=jax.ShapeDtypeStruct((NC, NPAD, 128), jnp.float32),
    mesh=_mesh,
    scratch_types=[
        pltpu.VMEM_SHARED((NPAD, 128), jnp.float32),
        pltpu.VMEM((RCH2,), jnp.int32),
        pltpu.VMEM((RCH2,), jnp.int32),
        pltpu.VMEM((RCH2,), jnp.int32),
        pltpu.VMEM((RCH2,), jnp.int32),
        pltpu.VMEM((RCH2, 128), jnp.float32),
        pltpu.VMEM((RCH2, 128), jnp.float32),
        pltpu.SemaphoreType.DMA,
        pltpu.SemaphoreType.DMA,
    ],
)

GCH = 160
GNT = EPT // GCH


def _gat2_body(h0, ed0, es0, h1, ed1, es1, sa0, da0, sa1, da1, zrows, z1d,
               outn, outd, accn, accd,
               ixs0, ixd0, rows0, edv0, esv0, exv0, gsem0,
               ixs1, ixd1, rows1, edv1, esv1, exv1, gsem1):
    cid = lax.axis_index("c")
    sid = lax.axis_index("s")
    pltpu.sync_copy(zrows.at[pl.ds(0, SPT)], accn.at[pl.ds(sid * SPT, SPT)])
    pltpu.sync_copy(z1d.at[pl.ds(0, SPT)], accd.at[pl.ds(sid * SPT, SPT)])
    plsc.subcore_barrier()

    def graph_loop(h, ed, es, src, dst):
        base0 = sid * EPT

        def issue(c, ixs, ixd, rows, edv, esv, sem):
            pltpu.sync_copy(src.at[pl.ds(base0 + c * GCH, GCH)], ixs)
            pltpu.sync_copy(dst.at[pl.ds(base0 + c * GCH, GCH)], ixd)
            pltpu.async_copy(es.at[ixs], esv, sem)
            pltpu.async_copy(ed.at[ixd], edv, sem)
            pltpu.async_copy(h.at[ixs], rows, sem)

        def finish(ixs, ixd, rows, edv, esv, exv, sem):
            pltpu.make_async_copy(es.at[ixs], esv, sem).wait()
            pltpu.make_async_copy(ed.at[ixd], edv, sem).wait()
            pltpu.make_async_copy(h.at[ixs], rows, sem).wait()
            for g in range(GCH // 16):
                e = edv[pl.ds(g * 16, 16)] + esv[pl.ds(g * 16, 16)]
                ex = jnp.exp(jnp.maximum(e, 0.2 * e))
                exv[pl.ds(g * 16, 16)] = ex
                for jj in range(16):
                    i = g * 16 + jj
                    b = jnp.full((16,), ex[jj])
                    for q in range(4):
                        rows[i, pl.ds(q * 16, 16)] = (
                            rows[i, pl.ds(q * 16, 16)] * b)
            pltpu.sync_copy(exv, accd.at[ixd], add=True)
            pltpu.sync_copy(rows, accn.at[ixd], add=True)

        issue(0, ixs0, ixd0, rows0, edv0, esv0, gsem0)

        def body(k, carry):
            c1 = 2 * k + 1
            c2 = 2 * k + 2

            @pl.when(c1 < GNT)
            def _():
                issue(c1, ixs1, ixd1, rows1, edv1, esv1, gsem1)

            finish(ixs0, ixd0, rows0, edv0, esv0, exv0, gsem0)

            @pl.when(c2 < GNT)
            def _():
                issue(c2, ixs0, ixd0, rows0, edv0, esv0, gsem0)

            @pl.when(c1 < GNT)
            def _():
                finish(ixs1, ixd1, rows1, edv1, esv1, exv1, gsem1)

            return carry

        lax.fori_loop(0, (GNT + 1) // 2, body, 0)

    @pl.when(cid == 0)
    def _():
        graph_loop(h0, ed0, es0, sa0, da0)

    @pl.when(cid == 1)
    def _():
        graph_loop(h1, ed1, es1, sa1, da1)

    plsc.subcore_barrier()
    pltpu.sync_copy(accn.at[pl.ds(sid * SPT, SPT)],
                    outn.at[cid, pl.ds(sid * SPT, SPT)])
    pltpu.sync_copy(accd.at[pl.ds(sid * SPT, SPT)],
                    outd.at[pl.ds(cid * NPAD + sid * SPT, SPT)])


_gat2_kernel = pl.kernel(
    _gat2_body,
    out_type=(jax.ShapeDtypeStruct((NC, NPAD, 128), jnp.float32),
              jax.ShapeDtypeStruct((NC * NPAD,), jnp.float32)),
    mesh=_mesh,
    scratch_types=[
        pltpu.VMEM_SHARED((NPAD, 128), jnp.float32),
        pltpu.VMEM_SHARED((NPAD,), jnp.float32),
        pltpu.VMEM((GCH,), jnp.int32),
        pltpu.VMEM((GCH,), jnp.int32),
        pltpu.VMEM((GCH, 128), jnp.float32),
        pltpu.VMEM((GCH,), jnp.float32),
        pltpu.VMEM((GCH,), jnp.float32),
        pltpu.VMEM((GCH,), jnp.float32),
        pltpu.SemaphoreType.DMA,
        pltpu.VMEM((GCH,), jnp.int32),
        pltpu.VMEM((GCH,), jnp.int32),
        pltpu.VMEM((GCH, 128), jnp.float32),
        pltpu.VMEM((GCH,), jnp.float32),
        pltpu.VMEM((GCH,), jnp.float32),
        pltpu.VMEM((GCH,), jnp.float32),
        pltpu.SemaphoreType.DMA,
    ],
)


def _prep_body(deg4, x, x1, xs, xs1, dinv0, dinv1, inv0, inv1):
    i0 = deg4[0] + deg4[1]
    i1 = deg4[2] + deg4[3]
    dv0 = 1.0 / jnp.sqrt(i0 + 1.0)
    dv1 = 1.0 / jnp.sqrt(i1 + 1.0)
    dinv0[...] = dv0
    dinv1[...] = dv1
    inv0[...] = 1.0 / jnp.maximum(i0, 1.0)
    inv1[...] = 1.0 / jnp.maximum(i1, 1.0)
    xs[:N, :] = x[...] * dv0[:N].reshape(N, 1)
    xs[N:, :] = jnp.zeros((NPAD - N, 128), jnp.float32)
    xs1[:N, :] = x1[...] * dv1[:N].reshape(N, 1)
    xs1[N:, :] = jnp.zeros((NPAD - N, 128), jnp.float32)


_prep_tc = pl.pallas_call(
    _prep_body,
    out_shape=(
        jax.ShapeDtypeStruct((NPAD, 128), jnp.float32),
        jax.ShapeDtypeStruct((NPAD, 128), jnp.float32),
        jax.ShapeDtypeStruct((NPAD,), jnp.float32),
        jax.ShapeDtypeStruct((NPAD,), jnp.float32),
        jax.ShapeDtypeStruct((NPAD,), jnp.float32),
        jax.ShapeDtypeStruct((NPAD,), jnp.float32),
    ),
)


def _gcn_mm_body(agg_in, xs, dinv, W, b, out):
    agg = agg_in[...] + xs[...]
    scaled = agg * dinv[...].reshape(NPAD, 1)
    out[...] = jnp.dot(scaled, W[...],
                       preferred_element_type=jnp.float32) + b[...]


_gcn_mm = pl.pallas_call(
    _gcn_mm_body,
    out_shape=jax.ShapeDtypeStruct((NPAD, 256), jnp.float32),
)


def _sagepre1_body(xa, xb, Wn, Ws, p, q):
    x0 = jax.nn.relu(xa[...]) + jax.nn.relu(xb[...])
    p[...] = jnp.dot(x0, Wn[...], preferred_element_type=jnp.float32)
    q[...] = jnp.dot(x0, Ws[...], preferred_element_type=jnp.float32)


def _sagepre2_body(xb, Wn, Ws, p, q):
    t = jax.nn.relu(xb[...]) + xb[...]
    p[...] = jnp.dot(t, Wn[...], preferred_element_type=jnp.float32)
    q[...] = jnp.dot(t, Ws[...], preferred_element_type=jnp.float32)


_sage_out = (jax.ShapeDtypeStruct((NPAD, 128), jnp.float32),
             jax.ShapeDtypeStruct((NPAD, 128), jnp.float32))
_sagepre1 = pl.pallas_call(_sagepre1_body, out_shape=_sage_out)
_sagepre2 = pl.pallas_call(_sagepre2_body, out_shape=_sage_out)


def _gatpre_body(agg_in, inv, q, bn, W, Wb, a, ab, h128, ed, es):
    xc = agg_in[...] * inv[...].reshape(NPAD, 1) + bn[...] + q[...]
    h = jnp.dot(xc, W[...], preferred_element_type=jnp.float32) + Wb[...]
    h128[:, :64] = h
    h128[:, 64:] = jnp.zeros((NPAD, 64), jnp.float32)
    ed[...] = jnp.dot(h, a[:64, 0], preferred_element_type=jnp.float32) + ab[...]
    es[...] = jnp.dot(h, a[64:, 0], preferred_element_type=jnp.float32)


_gatpre = pl.pallas_call(
    _gatpre_body,
    out_shape=(
        jax.ShapeDtypeStruct((NPAD, 128), jnp.float32),
        jax.ShapeDtypeStruct((NPAD,), jnp.float32),
        jax.ShapeDtypeStruct((NPAD,), jnp.float32),
    ),
)


def _final_body(nn, dd, W1, b1, g, bta, W2, b2, out):
    def branch(gi):
        num = nn[gi, :, :64]
        den = dd[gi]
        deni = jnp.where(den > 0, 1.0 / den, 0.0).reshape(NPAD, 1)
        xc = jax.nn.relu(num * deni)
        z = jnp.dot(xc, W1[...], preferred_element_type=jnp.float32) + b1[...]
        mu = jnp.mean(z, axis=-1, keepdims=True)
        var = jnp.mean((z - mu) * (z - mu), axis=-1, keepdims=True)
        z = (z - mu) / jnp.sqrt(var + 1e-5) * g[...] + bta[...]
        z = jnp.dot(z, W2[...], preferred_element_type=jnp.float32) + b2[...]
        return jnp.sum(z[:N, :])

    out[...] = (branch(0) + branch(1)).reshape(1, 1)


_final_tc = pl.pallas_call(
    _final_body,
    out_shape=jax.ShapeDtypeStruct((1, 1), jnp.float32),
)


def kernel(x, x1, edge_index, edge_index1, gcn1_W, gcn1_b, gcn2_W, gcn2_b,
           sage1_Wn, sage1_b, sage1_Ws, sage2_Wn, sage2_b, sage2_Ws,
           gat1_W, gat1_Wb, gat1_a, gat1_ab, gat2_W, gat2_Wb, gat2_a, gat2_ab,
           mlp_W1, mlp_b1, ln_g, ln_b, mlp_W2, mlp_b2):
    ei = edge_index.astype(jnp.int32)
    ei1 = edge_index1.astype(jnp.int32)
    s0, d0 = ei[0], ei[1]
    s1, d1 = ei1[0], ei1[1]

    ones_ch = jnp.ones((CH,), jnp.float32)
    zeros_1d = jnp.zeros((SPT,), jnp.float32)
    zeros_rows = jnp.zeros((SPT, 128), jnp.float32)

    deg4 = _deg_kernel(d0, d1, ones_ch, zeros_1d).reshape(2 * NC, NPAD)
    xs, xs1, dinv0, dinv1, inv0, inv1 = _prep_tc(deg4, x, x1)

    A = _rowagg2(xs, xs1, s0, d0, s1, d1, zeros_rows)
    xa = _gcn_mm(A[0], xs, dinv0, gcn1_W, gcn1_b)
    xb = _gcn_mm(A[1], xs1, dinv1, gcn2_W, gcn2_b)

    p1, q1 = _sagepre1(xa, xb, sage1_Wn, sage1_Ws)
    p2, q2 = _sagepre2(xb, sage2_Wn, sage2_Ws)
    P = _rowagg2(p1, p2, s0, d0, s1, d1, zeros_rows)

    h1, ed1, es1 = _gatpre(P[0], inv0, q1, sage1_b, gat1_W, gat1_Wb,
                           gat1_a, gat1_ab)
    h2, ed2, es2 = _gatpre(P[1], inv1, q2, sage2_b, gat2_W, gat2_Wb,
                           gat2_a, gat2_ab)

    nn, dd = _gat2_kernel(h1, ed1, es1, h2, ed2, es2, s0, d0, s1, d1,
                          zeros_rows, zeros_1d)
    total = _final_tc(nn, dd.reshape(NC, NPAD),
                      mlp_W1, mlp_b1, ln_g, ln_b, mlp_W2, mlp_b2)
    return total / (2 * N * 64)
